# baseline (speedup 1.0000x reference)
import functools

import jax
import jax.numpy as jnp
from jax import lax
from jax.experimental import pallas as pl
from jax.experimental.pallas import tpu as pltpu
from jax.experimental.pallas import tpu_sc as plsc

B, L, V, E, H = 16, 512, 1000, 128, 128
G4 = 4 * H
TCH = 64
NSTEP = L // TCH


@functools.cache
def _sc_gather_fn():
    mesh = plsc.VectorSubcoreMesh(core_axis_name="c", subcore_axis_name="s")
    nw = mesh.num_cores * mesh.num_subcores
    n = B * L
    assert n % (8 * nw) == 0
    bpw = n // nw

    @functools.partial(
        pl.kernel,
        mesh=mesh,
        out_type=jax.ShapeDtypeStruct((n, E), jnp.float32),
        scratch_types=[
            pltpu.VMEM((bpw,), jnp.int32),
            pltpu.VMEM((bpw, E), jnp.float32),
            pltpu.SemaphoreType.DMA,
        ],
    )
    def gather_k(table_hbm, idx_hbm, out_hbm, idx_v, rows_v, sem):
        wid = lax.axis_index("s") * mesh.num_cores + lax.axis_index("c")
        base = wid * bpw
        pltpu.sync_copy(idx_hbm.at[pl.ds(base, bpw)], idx_v)
        pltpu.async_copy(table_hbm.at[idx_v], rows_v, sem).wait()
        pltpu.sync_copy(rows_v, out_hbm.at[pl.ds(base, bpw)])

    return gather_k


def _rec_body(emb_ref, va_ref, q_ref, wihT_ref, whhT_ref, b_ref,
              out_ref, h_ref, c_ref, gx_ref):
    pid = pl.program_id(0)

    q = q_ref[...]
    nz = jnp.sum((q == 0).astype(jnp.int32), axis=1, keepdims=True)
    lengths = jnp.minimum(L - 1, L - nz)

    @pl.when(pid == 0)
    def _prime():
        g0 = jnp.dot(va_ref[...].astype(jnp.bfloat16), wihT_ref[...],
                     preferred_element_type=jnp.float32) + b_ref[...]
        i_ = jax.nn.sigmoid(g0[:, 0:H])
        g_ = jnp.tanh(g0[:, 2 * H:3 * H])
        o_ = jax.nn.sigmoid(g0[:, 3 * H:4 * H])
        c1 = i_ * g_
        h_ref[...] = o_ * jnp.tanh(c1)
        c_ref[...] = c1

    x = emb_ref[...].reshape(TCH * B, E).astype(jnp.bfloat16)
    gx = jnp.dot(x, wihT_ref[...], preferred_element_type=jnp.float32)
    gx_ref[...] = gx.reshape(TCH, B, G4) + b_ref[...]

    h, c = h_ref[...], c_ref[...]
    for t in range(TCH):
        gates = gx_ref[t] + jnp.dot(h, whhT_ref[...],
                                    preferred_element_type=jnp.float32)
        i_ = jax.nn.sigmoid(gates[:, 0:H])
        f_ = jax.nn.sigmoid(gates[:, H:2 * H])
        g_ = jnp.tanh(gates[:, 2 * H:3 * H])
        o_ = jax.nn.sigmoid(gates[:, 3 * H:4 * H])
        c2 = f_ * c + i_ * g_
        h2 = o_ * jnp.tanh(c2)
        m = (pid * TCH + t) < lengths
        out_ref[t] = jnp.where(m, h2, 0.0)
        h = jnp.where(m, h2, h)
        c = jnp.where(m, c2, c)
    h_ref[...] = h
    c_ref[...] = c


@jax.jit
def _rec_call(emb_tm, va_feat, q, wihT, whhT, bsum):
    return pl.pallas_call(
        _rec_body,
        grid=(NSTEP,),
        in_specs=[
            pl.BlockSpec((TCH, B, E), lambda i: (i, 0, 0)),
            pl.BlockSpec((B, E), lambda i: (0, 0)),
            pl.BlockSpec((B, L), lambda i: (0, 0)),
            pl.BlockSpec((E, G4), lambda i: (0, 0)),
            pl.BlockSpec((H, G4), lambda i: (0, 0)),
            pl.BlockSpec((1, G4), lambda i: (0, 0)),
        ],
        out_specs=pl.BlockSpec((TCH, B, H), lambda i: (i, 0, 0)),
        out_shape=jax.ShapeDtypeStruct((L, B, H), jnp.float32),
        scratch_shapes=[
            pltpu.VMEM((B, H), jnp.float32),
            pltpu.VMEM((B, H), jnp.float32),
            pltpu.VMEM((TCH, B, G4), jnp.float32),
        ],
        compiler_params=pltpu.CompilerParams(
            dimension_semantics=("arbitrary",),
        ),
    )(emb_tm, va_feat, q, wihT, whhT, bsum)


def _cls_body(f_ref, wclsT_ref, out_ref):
    fb = jnp.stack([f_ref[:, b, :] for b in range(B)])
    fb = fb.reshape(B * TCH, H).astype(jnp.bfloat16)
    y = jnp.dot(fb, wclsT_ref[...], preferred_element_type=jnp.float32)
    out_ref[...] = y.reshape(B, TCH, V)


@jax.jit
def _cls_call(feats_tm, wclsT):
    return pl.pallas_call(
        _cls_body,
        grid=(NSTEP,),
        in_specs=[
            pl.BlockSpec((TCH, B, H), lambda i: (i, 0, 0)),
            pl.BlockSpec((H, V), lambda i: (0, 0)),
        ],
        out_specs=pl.BlockSpec((B, TCH, V), lambda i: (0, i, 0)),
        out_shape=jax.ShapeDtypeStruct((B, L, V), jnp.float32),
        compiler_params=pltpu.CompilerParams(
            dimension_semantics=("arbitrary",),
        ),
    )(feats_tm, wclsT)


def kernel(va_feat, questions, embedding, W_ih, W_hh, b_ih, b_hh, W_cls):
    q = questions.astype(jnp.int32)
    idx = q.T.reshape(-1)
    emb_flat = _sc_gather_fn()(embedding.astype(jnp.float32), idx)
    emb_tm = emb_flat.reshape(L, B, E)
    bsum = (b_ih + b_hh).reshape(1, G4)
    feats_tm = _rec_call(emb_tm, va_feat, q, W_ih.T.astype(jnp.bfloat16),
                         W_hh.T, bsum)
    return _cls_call(feats_tm, W_cls.T.astype(jnp.bfloat16))


# speedup vs baseline: 1.0618x; 1.0618x over previous
import functools

import jax
import jax.numpy as jnp
from jax import lax
from jax.experimental import pallas as pl
from jax.experimental.pallas import tpu as pltpu
from jax.experimental.pallas import tpu_sc as plsc

B, L, V, E, H = 16, 512, 1000, 128, 128
G4 = 4 * H
TCH = 64
NSTEP = L // TCH

_TDIMS = (((1,), (1,)), ((), ()))


def _tdot(a, w):
    return lax.dot_general(a, w, _TDIMS, preferred_element_type=jnp.float32)


@functools.cache
def _sc_gather_fn():
    mesh = plsc.VectorSubcoreMesh(core_axis_name="c", subcore_axis_name="s")
    nw = mesh.num_cores * mesh.num_subcores
    n = B * L
    assert n % (8 * nw) == 0
    bpw = n // nw

    @functools.partial(
        pl.kernel,
        mesh=mesh,
        out_type=jax.ShapeDtypeStruct((n, E), jnp.float32),
        scratch_types=[
            pltpu.VMEM((bpw,), jnp.int32),
            pltpu.VMEM((bpw, E), jnp.float32),
            pltpu.SemaphoreType.DMA,
        ],
    )
    def gather_k(table_hbm, idx_hbm, out_hbm, idx_v, rows_v, sem):
        wid = lax.axis_index("s") * mesh.num_cores + lax.axis_index("c")
        base = wid * bpw
        pltpu.sync_copy(idx_hbm.at[pl.ds(base, bpw)], idx_v)
        pltpu.async_copy(table_hbm.at[idx_v], rows_v, sem).wait()
        pltpu.sync_copy(rows_v, out_hbm.at[pl.ds(base, bpw)])

    return gather_k


def _out_copy(ybuf_ref, out_ref, sem, slot, step):
    return pltpu.make_async_copy(
        ybuf_ref.at[slot],
        out_ref.at[:, pl.ds(step * TCH, TCH), :],
        sem.at[slot],
    )


def _tc_body(emb_ref, va_ref, q_ref, wih_ref, whh_ref, bih_ref, bhh_ref,
             wcls_ref, out_ref, h_ref, c_ref, gx_ref, ftm_ref, fbm_ref,
             ybuf_ref, sem):
    pid = pl.program_id(0)
    slot = lax.rem(pid, 2)

    q = q_ref[...]
    nz = jnp.sum((q == 0).astype(jnp.int32), axis=1, keepdims=True)
    lengths = jnp.minimum(L - 1, L - nz)

    bsum = bih_ref[...] + bhh_ref[...]

    @pl.when(pid == 0)
    def _prime():
        g0 = _tdot(va_ref[...], wih_ref[...]) + bsum
        i_ = jax.nn.sigmoid(g0[:, 0:H])
        g_ = jnp.tanh(g0[:, 2 * H:3 * H])
        o_ = jax.nn.sigmoid(g0[:, 3 * H:4 * H])
        c1 = i_ * g_
        h_ref[...] = o_ * jnp.tanh(c1)
        c_ref[...] = c1

    @pl.when(pid >= 2)
    def _drain():
        _out_copy(ybuf_ref, out_ref, sem, slot, pid - 2).wait()

    x = emb_ref[...].reshape(TCH * B, E)
    gx = _tdot(x, wih_ref[...])
    gx_ref[...] = gx.reshape(TCH, B, G4) + bsum

    h, c = h_ref[...], c_ref[...]
    for t in range(TCH):
        gates = gx_ref[t] + _tdot(h, whh_ref[...])
        i_ = jax.nn.sigmoid(gates[:, 0:H])
        f_ = jax.nn.sigmoid(gates[:, H:2 * H])
        g_ = jnp.tanh(gates[:, 2 * H:3 * H])
        o_ = jax.nn.sigmoid(gates[:, 3 * H:4 * H])
        c2 = f_ * c + i_ * g_
        h2 = o_ * jnp.tanh(c2)
        m = (pid * TCH + t) < lengths
        ftm_ref[t] = jnp.where(m, h2, 0.0)
        h = jnp.where(m, h2, h)
        c = jnp.where(m, c2, c)
    h_ref[...] = h
    c_ref[...] = c

    for b in range(B):
        fbm_ref[b] = ftm_ref[:, b, :]

    fb = fbm_ref[...].reshape(B * TCH, H)
    y = _tdot(fb, wcls_ref[...])
    ybuf_ref[slot] = y.reshape(B, TCH, V)
    _out_copy(ybuf_ref, out_ref, sem, slot, pid).start()

    @pl.when(pid == NSTEP - 1)
    def _final_drain():
        _out_copy(ybuf_ref, out_ref, sem, 1 - slot, pid - 1).wait()
        _out_copy(ybuf_ref, out_ref, sem, slot, pid).wait()


@jax.jit
def _tc_call(emb_tm, va_feat, q, wih, whh, bih, bhh, wcls):
    return pl.pallas_call(
        _tc_body,
        grid=(NSTEP,),
        in_specs=[
            pl.BlockSpec((TCH, B, E), lambda i: (i, 0, 0)),
            pl.BlockSpec((B, E), lambda i: (0, 0)),
            pl.BlockSpec((B, L), lambda i: (0, 0)),
            pl.BlockSpec((G4, E), lambda i: (0, 0)),
            pl.BlockSpec((G4, H), lambda i: (0, 0)),
            pl.BlockSpec((1, G4), lambda i: (0, 0)),
            pl.BlockSpec((1, G4), lambda i: (0, 0)),
            pl.BlockSpec((V, H), lambda i: (0, 0)),
        ],
        out_specs=pl.BlockSpec(memory_space=pl.ANY),
        out_shape=jax.ShapeDtypeStruct((B, L, V), jnp.float32),
        scratch_shapes=[
            pltpu.VMEM((B, H), jnp.float32),
            pltpu.VMEM((B, H), jnp.float32),
            pltpu.VMEM((TCH, B, G4), jnp.float32),
            pltpu.VMEM((TCH, B, H), jnp.float32),
            pltpu.VMEM((B, TCH, H), jnp.float32),
            pltpu.VMEM((2, B, TCH, V), jnp.float32),
            pltpu.SemaphoreType.DMA((2,)),
        ],
        compiler_params=pltpu.CompilerParams(
            dimension_semantics=("arbitrary",),
        ),
    )(emb_tm, va_feat, q, wih, whh, bih, bhh, wcls)


def kernel(va_feat, questions, embedding, W_ih, W_hh, b_ih, b_hh, W_cls):
    q = questions.astype(jnp.int32)
    idx = q.T.reshape(-1)
    emb_flat = _sc_gather_fn()(embedding.astype(jnp.float32), idx)
    emb_tm = emb_flat.reshape(L, B, E)
    return _tc_call(emb_tm, va_feat, q, W_ih, W_hh,
                    b_ih.reshape(1, G4), b_hh.reshape(1, G4), W_cls)
